# baseline (speedup 1.0000x reference)
import functools

import jax
import jax.numpy as jnp
from jax import lax
from jax.experimental import pallas as pl
from jax.experimental.pallas import tpu as pltpu
from jax.experimental.pallas import tpu_sc as plsc

_EPS = 0.5
_NC = 2
_NS = 16
_L = 16
_GPC = 2
_RIL = 8


def _make_sc_call(B, N):
    nw = _NC * _NS
    rows_per_w = B // nw
    groups = rows_per_w // _L
    nchunks = groups // _GPC
    chunk_rows = _GPC * _L
    rstride = N + 1
    halves = []
    h = 1
    while h < N:
        halves.append(h)
        h *= 2
    halves.reverse()

    mesh = plsc.VectorSubcoreMesh(
        core_axis_name="c", subcore_axis_name="s",
        num_cores=_NC, num_subcores=_NS)

    @functools.partial(
        pl.kernel,
        out_type=jax.ShapeDtypeStruct((B,), jnp.float32),
        mesh=mesh,
        compiler_params=pltpu.CompilerParams(needs_layout_passes=False),
        scratch_types=[
            pltpu.VMEM((chunk_rows, rstride), jnp.float32),
            pltpu.VMEM((chunk_rows * rstride,), jnp.float32),
            pltpu.VMEM((chunk_rows,), jnp.float32),
            pltpu.VMEM((rows_per_w,), jnp.float32),
            pltpu.VMEM_SHARED((_NS * chunk_rows, N), jnp.float32),
            pltpu.VMEM_SHARED((_NS * chunk_rows, N), jnp.float32),
            pltpu.SemaphoreType.DMA,
            pltpu.SemaphoreType.DMA,
        ],
    )
    def sc_call(theta_hbm, t_hbm, out_hbm, tbuf, cum_v, t_v, x_v,
                sp_a, sp_b, sem0, sem1):
        sps = (sp_a, sp_b)
        sems = (sem0, sem1)
        cid = lax.axis_index("c")
        sid = lax.axis_index("s")
        crow0 = cid * (_NS * rows_per_w)
        sc_chunk_rows = _NS * chunk_rows
        lane = lax.iota(jnp.int32, _L)
        idx15 = jnp.full((_L,), _L - 1, jnp.int32)

        def tile_rows0(ci):
            return crow0 + ci * sc_chunk_rows + sid * chunk_rows

        def sc_chunk_src(ci):
            return theta_hbm.at[pl.ds(crow0 + ci * sc_chunk_rows,
                                      sc_chunk_rows), :]

        def stage(par, ci):
            @pl.when(ci < nchunks)
            def _():
                @pl.when(sid == 0)
                def _():
                    pltpu.async_copy(sc_chunk_src(ci), sps[par], sems[par])

        stage(0, 0)
        stage(1, 1)

        def process_chunk(buf, ci):
            zf = jnp.zeros((_L,), jnp.float32)

            for b in range(chunk_rows // _RIL):
                rows_b = [b * _RIL + k for k in range(_RIL)]

                def pass1(m, cs, rows_b=rows_b):
                    cs = list(cs)
                    c0 = m * _L
                    for k, rr in enumerate(rows_b):
                        v = buf[rr, pl.ds(c0, _L)]
                        s = plsc.cumsum(v) + cs[k]
                        cum_v[pl.ds(rr * rstride + c0, _L)] = s
                        cs[k] = s.at[idx15].get(mode="promise_in_bounds")
                    return tuple(cs)

                lax.fori_loop(0, N // _L, pass1, (zf,) * _RIL)

            tvs = [t_v[pl.ds(u * _L, _L)] for u in range(_GPC)]
            rows = [u * _L + lane for u in range(_GPC)]
            rbase = [(u * _L + lane) * rstride for u in range(_GPC)]
            poss = [jnp.zeros((_L,), jnp.int32) for _ in range(_GPC)]
            for half in halves:
                vs = [plsc.load_gather(
                    cum_v, [rbase[u] + poss[u] + (half - 1)])
                    for u in range(_GPC)]
                poss = [jnp.where(vs[u] < tvs[u], poss[u] + half, poss[u])
                        for u in range(_GPC)]
            for u in range(_GPC):
                tv, pos = tvs[u], poss[u]
                v = plsc.load_gather(cum_v, [rbase[u] + pos])
                idx = pos + jnp.where(v < tv, 1, 0)
                idxc = jnp.minimum(idx, N - 1)
                j = jnp.maximum(idxc - 1, 0)
                cs_j = plsc.load_gather(cum_v, [rbase[u] + j])
                th_next = plsc.load_gather(buf, [rows[u], idxc])
                s1 = (tv - cs_j) / th_next
                jnf = idxc.astype(jnp.float32)
                jf = j.astype(jnp.float32)
                x_cand = jnf - _EPS + 2.0 * _EPS * s1
                x = jnp.where(jnp.logical_and(s1 == 0.0, j > 0),
                              jf - 1.0 + _EPS, x_cand)
                x_v[pl.ds(ci * chunk_rows + u * _L, _L)] = x

        def run_phase(par, ci):
            @pl.when(sid == 0)
            def _():
                pltpu.make_async_copy(sc_chunk_src(0), sps[par],
                                      sems[par]).wait()

            plsc.subcore_barrier()
            pltpu.sync_copy(t_hbm.at[pl.ds(tile_rows0(ci), chunk_rows)], t_v)
            pltpu.sync_copy(
                sps[par].at[pl.ds(sid * chunk_rows, chunk_rows), :],
                tbuf.at[:, pl.ds(0, N)])
            plsc.subcore_barrier()
            stage(par, ci + 2)
            process_chunk(tbuf, ci)

        def chunkpair(cp, _):
            for par in range(2):
                run_phase(par, 2 * cp + par)
            return 0

        lax.fori_loop(0, nchunks // 2, chunkpair, 0)
        for ci in range(nchunks):
            pltpu.async_copy(
                x_v.at[pl.ds(ci * chunk_rows, chunk_rows)],
                out_hbm.at[pl.ds(tile_rows0(ci), chunk_rows)], sem0)
        for ci in range(nchunks):
            pltpu.make_async_copy(
                x_v.at[pl.ds(0, chunk_rows)],
                out_hbm.at[pl.ds(0, chunk_rows)], sem0).wait()

    return sc_call


@jax.jit
def kernel(theta, t, Y_train):
    B, N = theta.shape
    del Y_train
    return _make_sc_call(B, N)(theta, t)


# speedup vs baseline: 1.2432x; 1.2432x over previous
import functools

import jax
import jax.numpy as jnp
from jax import lax
from jax.experimental import pallas as pl
from jax.experimental.pallas import tpu as pltpu
from jax.experimental.pallas import tpu_sc as plsc

_EPS = 0.5
_NC = 2
_NS = 16
_L = 16
_GPC = 2


def _make_sc_call(B, N):
    nw = _NC * _NS
    rows_per_w = B // nw
    groups = rows_per_w // _L
    nchunks = groups // _GPC
    chunk_rows = _GPC * _L
    rstride = N + 1
    halves = []
    h = 1
    while h < N:
        halves.append(h)
        h *= 2
    halves.reverse()

    mesh = plsc.VectorSubcoreMesh(
        core_axis_name="c", subcore_axis_name="s",
        num_cores=_NC, num_subcores=_NS)

    @functools.partial(
        pl.kernel,
        out_type=jax.ShapeDtypeStruct((B,), jnp.float32),
        mesh=mesh,
        compiler_params=pltpu.CompilerParams(needs_layout_passes=False),
        scratch_types=[
            pltpu.VMEM((chunk_rows, rstride), jnp.float32),
            pltpu.VMEM((_GPC * N * _L,), jnp.float32),
            pltpu.VMEM((chunk_rows,), jnp.float32),
            pltpu.VMEM((rows_per_w,), jnp.float32),
            pltpu.VMEM_SHARED((_NS * chunk_rows, N), jnp.float32),
            pltpu.VMEM_SHARED((_NS * chunk_rows, N), jnp.float32),
            pltpu.SemaphoreType.DMA,
            pltpu.SemaphoreType.DMA,
        ],
    )
    def sc_call(theta_hbm, t_hbm, out_hbm, tbuf, cum_v, t_v, x_v,
                sp_a, sp_b, sem0, sem1):
        sps = (sp_a, sp_b)
        sems = (sem0, sem1)
        cid = lax.axis_index("c")
        sid = lax.axis_index("s")
        crow0 = cid * (_NS * rows_per_w)
        sc_chunk_rows = _NS * chunk_rows
        lane = lax.iota(jnp.int32, _L)

        def tile_rows0(ci):
            return crow0 + ci * sc_chunk_rows + sid * chunk_rows

        def sc_chunk_src(ci):
            return theta_hbm.at[pl.ds(crow0 + ci * sc_chunk_rows,
                                      sc_chunk_rows), :]

        def stage(par, ci):
            @pl.when(ci < nchunks)
            def _():
                @pl.when(sid == 0)
                def _():
                    pltpu.async_copy(sc_chunk_src(ci), sps[par], sems[par])

        stage(0, 0)
        stage(1, 1)

        def process_chunk(buf, ci):
            unroll = 16
            rows = [u * _L + lane for u in range(_GPC)]
            zf = jnp.zeros((_L,), jnp.float32)

            def pass1(m, carry):
                cs = list(carry[:_GPC])
                col = carry[_GPC]
                ths = list(carry[_GPC + 1:])
                for uu in range(unroll):
                    ncol = col + 1
                    nths = [plsc.load_gather(buf, [rows[u], ncol])
                            for u in range(_GPC)]
                    for u in range(_GPC):
                        cs[u] = cs[u] + ths[u]
                        cum_v[pl.ds(u * (N * _L) + m * (unroll * _L)
                                    + uu * _L, _L)] = cs[u]
                    col, ths = ncol, nths
                return tuple(cs) + (col,) + tuple(ths)

            col0 = jnp.zeros((_L,), jnp.int32)
            th0 = [plsc.load_gather(buf, [rows[u], col0])
                   for u in range(_GPC)]
            lax.fori_loop(0, N // unroll, pass1,
                          (zf,) * _GPC + (col0,) + tuple(th0))

            tvs = [t_v[pl.ds(u * _L, _L)] for u in range(_GPC)]
            poss = [jnp.zeros((_L,), jnp.int32) for _ in range(_GPC)]
            cbase = [u * (N * _L) for u in range(_GPC)]
            for half in halves:
                vs = [plsc.load_gather(
                    cum_v, [cbase[u] + (poss[u] + (half - 1)) * _L + lane])
                    for u in range(_GPC)]
                poss = [jnp.where(vs[u] < tvs[u], poss[u] + half, poss[u])
                        for u in range(_GPC)]
            for u in range(_GPC):
                tv, pos = tvs[u], poss[u]
                v = plsc.load_gather(cum_v, [cbase[u] + pos * _L + lane])
                idx = pos + jnp.where(v < tv, 1, 0)
                idxc = jnp.minimum(idx, N - 1)
                j = jnp.maximum(idxc - 1, 0)
                cs_j = plsc.load_gather(cum_v, [cbase[u] + j * _L + lane])
                th_next = plsc.load_gather(buf, [rows[u], idxc])
                s1 = (tv - cs_j) / th_next
                jnf = idxc.astype(jnp.float32)
                jf = j.astype(jnp.float32)
                x_cand = jnf - _EPS + 2.0 * _EPS * s1
                x = jnp.where(jnp.logical_and(s1 == 0.0, j > 0),
                              jf - 1.0 + _EPS, x_cand)
                x_v[pl.ds(ci * chunk_rows + u * _L, _L)] = x

        def run_phase(par, ci):
            @pl.when(sid == 0)
            def _():
                pltpu.make_async_copy(sc_chunk_src(0), sps[par],
                                      sems[par]).wait()

            plsc.subcore_barrier()
            pltpu.sync_copy(t_hbm.at[pl.ds(tile_rows0(ci), chunk_rows)], t_v)
            pltpu.sync_copy(
                sps[par].at[pl.ds(sid * chunk_rows, chunk_rows), :],
                tbuf.at[:, pl.ds(0, N)])
            plsc.subcore_barrier()
            stage(par, ci + 2)
            process_chunk(tbuf, ci)

        def chunkpair(cp, _):
            for par in range(2):
                run_phase(par, 2 * cp + par)
            return 0

        lax.fori_loop(0, nchunks // 2, chunkpair, 0)
        for ci in range(nchunks):
            pltpu.async_copy(
                x_v.at[pl.ds(ci * chunk_rows, chunk_rows)],
                out_hbm.at[pl.ds(tile_rows0(ci), chunk_rows)], sem0)
        for ci in range(nchunks):
            pltpu.make_async_copy(
                x_v.at[pl.ds(0, chunk_rows)],
                out_hbm.at[pl.ds(0, chunk_rows)], sem0).wait()

    return sc_call


@jax.jit
def kernel(theta, t, Y_train):
    B, N = theta.shape
    del Y_train
    return _make_sc_call(B, N)(theta, t)
